# baseline (speedup 1.0000x reference)
import functools

import jax
import jax.numpy as jnp
from jax import lax
from jax.experimental import pallas as pl
from jax.experimental.pallas import tpu as pltpu
from jax.experimental.pallas import tpu_sc as plsc

_B, _L, _D, _V = 4096, 200, 64, 1000000
_NC, _NS = 2, 16
_NW = _NC * _NS
_SPW = _B // _NW
_CHUNK = 104
_W = _D // 2
_NREG = _W // 16

_HI = jnp.uint32(0xFFFF0000)


def _sc_bow_body(x_hbm, table_hbm, out_hbm, idx_raw, rows_v, out_buf,
                 sem0, sem1):
    wid = lax.axis_index("s") * _NC + lax.axis_index("c")
    base = wid * _SPW
    pltpu.sync_copy(x_hbm.at[pl.ds(base, _SPW)], idx_raw)

    def descs(s, buf_ref, sem):
        a = pltpu.make_async_copy(
            table_hbm.at[idx_raw.at[s, pl.ds(0, _CHUNK)]],
            buf_ref.at[pl.ds(0, _CHUNK)], sem)
        b = pltpu.make_async_copy(
            table_hbm.at[idx_raw.at[s, pl.ds(_CHUNK, _L - _CHUNK)]],
            buf_ref.at[pl.ds(_CHUNK, _L - _CHUNK)], sem)
        return a, b

    def issue(s, buf_ref, sem):
        a, b = descs(s, buf_ref, sem)
        a.start()
        b.start()

    def drain(s, buf_ref, sem):
        a, b = descs(s, buf_ref, sem)
        a.wait()
        b.wait()

    def accum(s, buf_ref):
        def row_body(j, accs):
            new = []
            for d in range(_NREG):
                u = buf_ref[j, pl.ds(d * 16, 16)]
                ev = lax.bitcast_convert_type(
                    lax.shift_left(u, jnp.uint32(16)), jnp.float32)
                od = lax.bitcast_convert_type(u & _HI, jnp.float32)
                new.append(accs[2 * d] + ev)
                new.append(accs[2 * d + 1] + od)
            return tuple(new)

        z = jnp.zeros((16,), jnp.float32)
        accs = lax.fori_loop(0, _L, row_body, (z,) * (2 * _NREG), unroll=8)
        for r in range(2 * _NREG):
            out_buf[s, pl.ds(r * 16, 16)] = accs[r]

    buf0, buf1 = rows_v.at[0], rows_v.at[1]
    issue(0, buf0, sem0)

    def pair_body(g, carry):
        s0 = g * 2
        issue(s0 + 1, buf1, sem1)
        drain(s0, buf0, sem0)
        accum(s0, buf0)

        @pl.when(g < _SPW // 2 - 1)
        def _():
            issue(s0 + 2, buf0, sem0)

        drain(s0 + 1, buf1, sem1)
        accum(s0 + 1, buf1)
        return carry

    lax.fori_loop(0, _SPW // 2, pair_body, 0)
    pltpu.sync_copy(out_buf, out_hbm.at[pl.ds(base, _SPW)])


def _sc_bow(x, table_u32):
    mesh = plsc.VectorSubcoreMesh(core_axis_name="c", subcore_axis_name="s")
    return pl.kernel(
        _sc_bow_body,
        mesh=mesh,
        compiler_params=pltpu.CompilerParams(use_tc_tiling_on_sc=False),
        out_type=jax.ShapeDtypeStruct((_B, _D), jnp.float32),
        scratch_types=[
            pltpu.VMEM((_SPW, _L), jnp.int32),
            pltpu.VMEM((2, _L, _W), jnp.uint32),
            pltpu.VMEM((_SPW, _D), jnp.float32),
            pltpu.SemaphoreType.DMA,
            pltpu.SemaphoreType.DMA,
        ],
    )(x, table_u32)


def _tc_head_body(bow_ref, t_ref, W_hT_ref, b_h_ref, gamma_ref, beta_ref,
                  W_o_ref, b_o_ref, loss_ref, logits_ref):
    bow = bow_ref[...] * (1.0 / _L)
    h_lin = jnp.dot(bow, W_hT_ref[...],
                    preferred_element_type=jnp.float32) + b_h_ref[...]
    mu = jnp.mean(h_lin, axis=0, keepdims=True)
    xc = h_lin - mu
    var = jnp.mean(xc * xc, axis=0, keepdims=True)
    h = xc * lax.rsqrt(var + 1e-5) * gamma_ref[...] + beta_ref[...]
    h = jnp.maximum(h, 0.0)
    logit = (jnp.sum(h * W_o_ref[...], axis=1, keepdims=True)
             + b_o_ref[...])
    t = t_ref[...]
    per = (jnp.maximum(logit, 0.0) - logit * t
           + jnp.log1p(jnp.exp(-jnp.abs(logit))))
    loss_ref[...] = jnp.mean(per, keepdims=True)
    logits_ref[...] = logit


def _tc_head(bow_sum, t, W_h, b_h, gamma, beta, W_o, b_o):
    return pl.pallas_call(
        _tc_head_body,
        out_shape=(jax.ShapeDtypeStruct((1, 1), jnp.float32),
                   jax.ShapeDtypeStruct((_B, 1), jnp.float32)),
    )(bow_sum, t.reshape(_B, 1), W_h.T, b_h.reshape(1, _D),
      gamma.reshape(1, _D), beta.reshape(1, _D), W_o, b_o.reshape(1, 1))


def kernel(x, t, table, W_h, b_h, gamma, beta, W_o, b_o):
    table_u32 = lax.bitcast_convert_type(
        table.astype(jnp.bfloat16).reshape(_V, _W, 2), jnp.uint32)
    bow_perm = _sc_bow(x.astype(jnp.int32), table_u32)
    bow_sum = (bow_perm.reshape(_B, _NREG, 2, 16)
               .swapaxes(2, 3).reshape(_B, _D))
    loss2d, logits2d = _tc_head(bow_sum, t, W_h, b_h, gamma, beta, W_o, b_o)
    return loss2d[0, 0], logits2d[:, 0]


# speedup vs baseline: 2.1956x; 2.1956x over previous
import functools

import jax
import jax.numpy as jnp
from jax import lax
from jax.experimental import pallas as pl
from jax.experimental.pallas import tpu as pltpu
from jax.experimental.pallas import tpu_sc as plsc

_B, _L, _D, _V = 4096, 200, 64, 1000000
_NC, _NS = 2, 16
_NW = _NC * _NS
_SPW = _B // _NW
_CHUNK = 104
_NREG = _D // 16


def _sc_bow_body(x_hbm, table_hbm, out_hbm, idx_raw, rows_v, out_buf,
                 sem0, sem1):
    wid = lax.axis_index("s") * _NC + lax.axis_index("c")
    base = wid * _SPW
    pltpu.sync_copy(x_hbm.at[pl.ds(base, _SPW)], idx_raw)

    _C2 = _L - _CHUNK

    def desc(s, c, buf_ref, sem):
        off, n = (0, _CHUNK) if c == 0 else (_CHUNK, _C2)
        return pltpu.make_async_copy(
            table_hbm.at[idx_raw.at[s, pl.ds(off, n)]],
            buf_ref.at[pl.ds(0, n)], sem)

    def accum(buf_ref, n, accs):
        def row_body(j, a):
            return tuple(a[d] + buf_ref[j, pl.ds(d * 16, 16)]
                         for d in range(_NREG))
        return lax.fori_loop(0, n, row_body, accs, unroll=8)

    buf0, buf1 = rows_v.at[0], rows_v.at[1]
    desc(0, 0, buf0, sem0).start()

    def seq_body(s, carry):
        z = jnp.zeros((16,), jnp.float32)
        desc(s, 1, buf1, sem1).start()
        desc(s, 0, buf0, sem0).wait()
        accs = accum(buf0, _CHUNK, (z,) * _NREG)

        @pl.when(s < _SPW - 1)
        def _():
            desc(s + 1, 0, buf0, sem0).start()

        desc(s, 1, buf1, sem1).wait()
        accs = accum(buf1, _C2, accs)
        for d in range(_NREG):
            out_buf[s, pl.ds(d * 16, 16)] = accs[d]
        return carry

    lax.fori_loop(0, _SPW, seq_body, 0)
    pltpu.sync_copy(out_buf, out_hbm.at[pl.ds(base, _SPW)])


def _sc_bow(x, table):
    mesh = plsc.VectorSubcoreMesh(core_axis_name="c", subcore_axis_name="s")
    return pl.kernel(
        _sc_bow_body,
        mesh=mesh,
        compiler_params=pltpu.CompilerParams(use_tc_tiling_on_sc=False),
        out_type=jax.ShapeDtypeStruct((_B, _D), jnp.float32),
        scratch_types=[
            pltpu.VMEM((_SPW, _L), jnp.int32),
            pltpu.VMEM((2, _CHUNK, 2 * _D), jnp.float32),
            pltpu.VMEM((_SPW, _D), jnp.float32),
            pltpu.SemaphoreType.DMA,
            pltpu.SemaphoreType.DMA,
        ],
    )(x, table)


def _tc_head_body(bow_ref, t_ref, W_hT_ref, b_h_ref, gamma_ref, beta_ref,
                  W_o_ref, b_o_ref, loss_ref, logits_ref):
    bow = bow_ref[...] * (1.0 / _L)
    h_lin = jnp.dot(bow, W_hT_ref[...],
                    preferred_element_type=jnp.float32) + b_h_ref[...]
    mu = jnp.mean(h_lin, axis=0, keepdims=True)
    xc = h_lin - mu
    var = jnp.mean(xc * xc, axis=0, keepdims=True)
    h = xc * lax.rsqrt(var + 1e-5) * gamma_ref[...] + beta_ref[...]
    h = jnp.maximum(h, 0.0)
    logit = (jnp.sum(h * W_o_ref[...], axis=1, keepdims=True)
             + b_o_ref[...])
    t = t_ref[...]
    per = (jnp.maximum(logit, 0.0) - logit * t
           + jnp.log1p(jnp.exp(-jnp.abs(logit))))
    loss_ref[...] = jnp.mean(per, keepdims=True)
    logits_ref[...] = logit


def _tc_head(bow_sum, t, W_h, b_h, gamma, beta, W_o, b_o):
    return pl.pallas_call(
        _tc_head_body,
        out_shape=(jax.ShapeDtypeStruct((1, 1), jnp.float32),
                   jax.ShapeDtypeStruct((_B, 1), jnp.float32)),
    )(bow_sum, t.reshape(_B, 1), W_h.T, b_h.reshape(1, _D),
      gamma.reshape(1, _D), beta.reshape(1, _D), W_o, b_o.reshape(1, 1))


def kernel(x, t, table, W_h, b_h, gamma, beta, W_o, b_o):
    table_pad = jnp.pad(table, ((0, 0), (0, _D)))
    bow_sum = _sc_bow(x.astype(jnp.int32), table_pad)
    loss2d, logits2d = _tc_head(bow_sum, t, W_h, b_h, gamma, beta, W_o, b_o)
    return loss2d[0, 0], logits2d[:, 0]


# speedup vs baseline: 2.3407x; 1.0661x over previous
import jax
import jax.numpy as jnp
from jax import lax
from jax.experimental import pallas as pl
from jax.experimental.pallas import tpu as pltpu
from jax.experimental.pallas import tpu_sc as plsc

_B, _L, _D, _V = 4096, 200, 64, 1000000
_NC, _NS = 2, 16
_NW = _NC * _NS
_SPW = _B // _NW
_CHUNK = 104
_NREG = _D // 16


def _sc_bow_body(x_hbm, table_hbm, out_hbm, idx_raw, rows_v, out_buf,
                 sem0, sem1):
    wid = lax.axis_index("s") * _NC + lax.axis_index("c")
    base = wid * _SPW
    pltpu.sync_copy(x_hbm.at[pl.ds(base, _SPW)], idx_raw)

    def descs(s, buf_ref, sem):
        a = pltpu.make_async_copy(
            table_hbm.at[idx_raw.at[s, pl.ds(0, _CHUNK)]],
            buf_ref.at[pl.ds(0, _CHUNK)], sem)
        b = pltpu.make_async_copy(
            table_hbm.at[idx_raw.at[s, pl.ds(_CHUNK, _L - _CHUNK)]],
            buf_ref.at[pl.ds(_CHUNK, _L - _CHUNK)], sem)
        return a, b

    def issue(s, buf_ref, sem):
        a, b = descs(s, buf_ref, sem)
        a.start()
        b.start()

    def drain(s, buf_ref, sem):
        a, b = descs(s, buf_ref, sem)
        a.wait()
        b.wait()

    def accum(s, buf_ref):
        def row_body(j, accs):
            return tuple(accs[d] + buf_ref[j, pl.ds(d * 16, 16)]
                         for d in range(_NREG))
        z = jnp.zeros((16,), jnp.float32)
        accs = lax.fori_loop(0, _L, row_body, (z,) * _NREG, unroll=8)
        for d in range(_NREG):
            out_buf[s, pl.ds(d * 16, 16)] = accs[d]

    buf0, buf1 = rows_v.at[0], rows_v.at[1]
    issue(0, buf0, sem0)

    def pair_body(g, carry):
        s0 = g * 2
        issue(s0 + 1, buf1, sem1)
        drain(s0, buf0, sem0)
        accum(s0, buf0)

        @pl.when(g < _SPW // 2 - 1)
        def _():
            issue(s0 + 2, buf0, sem0)

        drain(s0 + 1, buf1, sem1)
        accum(s0 + 1, buf1)
        return carry

    lax.fori_loop(0, _SPW // 2, pair_body, 0)
    pltpu.sync_copy(out_buf, out_hbm.at[pl.ds(base, _SPW)])


def _sc_bow(x, table):
    mesh = plsc.VectorSubcoreMesh(core_axis_name="c", subcore_axis_name="s")
    return pl.kernel(
        _sc_bow_body,
        mesh=mesh,
        compiler_params=pltpu.CompilerParams(use_tc_tiling_on_sc=False),
        out_type=jax.ShapeDtypeStruct((_B, _D), jnp.float32),
        scratch_types=[
            pltpu.VMEM((_SPW, _L), jnp.int32),
            pltpu.VMEM((2, _L, _D), jnp.float32),
            pltpu.VMEM((_SPW, _D), jnp.float32),
            pltpu.SemaphoreType.DMA,
            pltpu.SemaphoreType.DMA,
        ],
    )(x, table)


def _tc_head_body(bow_ref, t_ref, W_hT_ref, b_h_ref, gamma_ref, beta_ref,
                  W_o_ref, b_o_ref, loss_ref, logits_ref):
    bow = bow_ref[...] * (1.0 / _L)
    h_lin = jnp.dot(bow, W_hT_ref[...],
                    preferred_element_type=jnp.float32) + b_h_ref[...]
    mu = jnp.mean(h_lin, axis=0, keepdims=True)
    xc = h_lin - mu
    var = jnp.mean(xc * xc, axis=0, keepdims=True)
    h = xc * lax.rsqrt(var + 1e-5) * gamma_ref[...] + beta_ref[...]
    h = jnp.maximum(h, 0.0)
    logit = (jnp.sum(h * W_o_ref[...], axis=1, keepdims=True)
             + b_o_ref[...])
    t = t_ref[...]
    per = (jnp.maximum(logit, 0.0) - logit * t
           + jnp.log1p(jnp.exp(-jnp.abs(logit))))
    loss_ref[...] = jnp.mean(per, keepdims=True)
    logits_ref[...] = logit


def _tc_head(bow_sum, t, W_h, b_h, gamma, beta, W_o, b_o):
    return pl.pallas_call(
        _tc_head_body,
        out_shape=(jax.ShapeDtypeStruct((1, 1), jnp.float32),
                   jax.ShapeDtypeStruct((_B, 1), jnp.float32)),
    )(bow_sum, t.reshape(_B, 1), W_h.T, b_h.reshape(1, _D),
      gamma.reshape(1, _D), beta.reshape(1, _D), W_o, b_o.reshape(1, 1))


def kernel(x, t, table, W_h, b_h, gamma, beta, W_o, b_o):
    bow_sum = _sc_bow(x.astype(jnp.int32), table)
    loss2d, logits2d = _tc_head(bow_sum, t, W_h, b_h, gamma, beta, W_o, b_o)
    return loss2d[0, 0], logits2d[:, 0]
